# baseline (speedup 1.0000x reference)
import functools

import jax
import jax.numpy as jnp
from jax import lax
from jax.experimental import pallas as pl
from jax.experimental.pallas import tpu as pltpu
from jax.experimental.pallas import tpu_sc as plsc

_NC = 2
_NS = 16
_CH = 80
_KI = 16
_KO = 8
_NBUF = 4
_PAD = 64


@functools.cache
def _make_deg(n, e):
    nw = _NC * _NS
    per_tile = e // nw
    nch = per_tile // _CH
    npad = ((n + nw * 16 - 1) // (nw * 16)) * (nw * 16)
    zseg = npad // _NS
    wb = 1000
    nwb = n // wb
    mesh = plsc.VectorSubcoreMesh(core_axis_name="c", subcore_axis_name="s")

    @functools.partial(
        pl.kernel,
        out_type=jax.ShapeDtypeStruct((_NC * n,), jnp.float32),
        mesh=mesh,
        scratch_types=[
            pltpu.VMEM((_KO, _KI, _CH), jnp.int32),
            pltpu.VMEM((_CH,), jnp.float32),
            pltpu.VMEM((zseg,), jnp.float32),
            pltpu.VMEM((wb,), jnp.float32),
            pltpu.VMEM_SHARED((npad,), jnp.float32),
        ],
    )
    def deg(dst_hbm, out_hbm, dst_v, ones_v, zbuf, wbuf, acc):
        c = lax.axis_index("c")
        s = lax.axis_index("s")
        w = s * _NC + c
        for j in range(_CH // 16):
            ones_v[pl.ds(j * 16, 16)] = jnp.ones((16,), jnp.float32)

        def zel(i, carry):
            zbuf[pl.ds(i * 16, 16)] = jnp.zeros((16,), jnp.float32)
            return carry

        lax.fori_loop(0, zseg // 16, zel, 0)
        pltpu.sync_copy(zbuf, acc.at[pl.ds(s * zseg, zseg)])
        plsc.subcore_barrier()
        pltpu.sync_copy(dst_hbm.at[w], dst_v)

        def body(k, carry):
            def inner(j, carry2):
                pltpu.sync_copy(ones_v, acc.at[dst_v.at[k, j]], add=True)
                return carry2

            return lax.fori_loop(0, _KI, inner, carry)

        lax.fori_loop(0, _KO, body, 0)
        plsc.subcore_barrier()

        @pl.when(s < nwb)
        def _():
            pltpu.sync_copy(acc.at[pl.ds(s * wb, wb)], wbuf)
            pltpu.sync_copy(wbuf, out_hbm.at[pl.ds(c * n + s * wb, wb)])

    return deg


@functools.cache
def _make_agg(n, e, f):
    zr = 40
    wb = 1000
    nwb = n // wb
    mesh = plsc.VectorSubcoreMesh(core_axis_name="c", subcore_axis_name="s")

    scratch = [
        pltpu.VMEM((1, _KI, _CH), jnp.int32),
        pltpu.VMEM((1, _KI, _CH), jnp.int32),
    ]
    scratch += [pltpu.VMEM((_CH, f), jnp.float32) for _ in range(_NBUF)]
    scratch += [
        pltpu.VMEM((zr, f), jnp.float32),
        pltpu.VMEM_SHARED((n + _PAD, f), jnp.float32),
    ]
    scratch += [pltpu.SemaphoreType.DMA for _ in range(_NBUF)]

    @functools.partial(
        pl.kernel,
        out_type=jax.ShapeDtypeStruct((_NC, n, f), jnp.float32),
        mesh=mesh,
        scratch_types=scratch,
    )
    def agg(hp_hbm, src_hbm, dst_hbm, out_hbm, src_v, dst_v, r0, r1, r2, r3,
            zbuf, acc, g0, g1, g2, g3):
        rows = (r0, r1, r2, r3)
        gsem = (g0, g1, g2, g3)
        c = lax.axis_index("c")
        s = lax.axis_index("s")
        w = s * _NC + c

        def zrow(r, carry):
            for j in range(f // 16):
                zbuf[r, pl.ds(j * 16, 16)] = jnp.zeros((16,), jnp.float32)
            return carry

        lax.fori_loop(0, zr, zrow, 0)

        @pl.when(s < nwb)
        def _():
            def zcp(i, carry):
                pltpu.sync_copy(zbuf, acc.at[pl.ds(s * wb + i * zr, zr)])
                return carry

            lax.fori_loop(0, wb // zr, zcp, 0)

        plsc.subcore_barrier()

        ngrp = _KI // _NBUF

        def blk(k, carry):
            pltpu.sync_copy(src_hbm.at[w, pl.ds(k, 1)], src_v)
            pltpu.sync_copy(dst_hbm.at[w, pl.ds(k, 1)], dst_v)
            for b in range(_NBUF):
                pltpu.async_copy(hp_hbm.at[src_v.at[0, jnp.int32(b)]],
                                 rows[b], gsem[b])

            def body(i, carry2):
                for b in range(_NBUF):
                    j = _NBUF * i + b
                    pltpu.make_async_copy(hp_hbm.at[src_v.at[0, j]], rows[b],
                                          gsem[b]).wait()
                    pltpu.sync_copy(rows[b], acc.at[dst_v.at[0, j]], add=True)
                    pltpu.async_copy(hp_hbm.at[src_v.at[0, j + _NBUF]],
                                     rows[b], gsem[b])
                return carry2

            lax.fori_loop(0, ngrp - 1, body, 0)
            for b in range(_NBUF):
                j = jnp.int32(_KI - _NBUF + b)
                pltpu.make_async_copy(hp_hbm.at[src_v.at[0, j]], rows[b],
                                      gsem[b]).wait()
                pltpu.sync_copy(rows[b], acc.at[dst_v.at[0, j]], add=True)
            return carry

        lax.fori_loop(0, _KO, blk, 0)

        plsc.subcore_barrier()

        @pl.when(s < nwb)
        def _():
            def wcp(i, carry):
                pltpu.sync_copy(acc.at[pl.ds(s * wb + i * zr, zr)], zbuf)
                pltpu.sync_copy(zbuf, out_hbm.at[c, pl.ds(s * wb + i * zr, zr)])
                return carry

            lax.fori_loop(0, wb // zr, wcp, 0)

    return agg


_BM = 1000


def _tc1(x, w1, d0, d1):
    n, f = x.shape

    def body(x_ref, w_ref, d0_ref, d1_ref, hp_ref, dinv_ref):
        dinv = lax.rsqrt(d0_ref[...] + d1_ref[...] + 1.0)
        h = jnp.dot(x_ref[...], w_ref[...], preferred_element_type=jnp.float32)
        hp_ref[...] = h * dinv
        dinv_ref[...] = dinv

    return pl.pallas_call(
        body,
        grid=(n // _BM,),
        in_specs=[
            pl.BlockSpec((_BM, f), lambda i: (i, 0)),
            pl.BlockSpec((f, f), lambda i: (0, 0)),
            pl.BlockSpec((_BM, 1), lambda i: (i, 0)),
            pl.BlockSpec((_BM, 1), lambda i: (i, 0)),
        ],
        out_specs=[
            pl.BlockSpec((_BM, f), lambda i: (i, 0)),
            pl.BlockSpec((_BM, 1), lambda i: (i, 0)),
        ],
        out_shape=[
            jax.ShapeDtypeStruct((n, f), jnp.float32),
            jax.ShapeDtypeStruct((n, 1), jnp.float32),
        ],
    )(x, w1, d0, d1)


def _tc2(p0, p1, hp1, dinv, b1, w2):
    n, f = hp1.shape

    def body(p0_ref, p1_ref, hp_ref, dinv_ref, b_ref, w_ref, out_ref):
        q = p0_ref[...] + p1_ref[...] + hp_ref[...]
        z = jnp.maximum(dinv_ref[...] * q + b_ref[...], 0.0)
        out_ref[...] = dinv_ref[...] * jnp.dot(
            z, w_ref[...], preferred_element_type=jnp.float32)

    return pl.pallas_call(
        body,
        grid=(n // _BM,),
        in_specs=[
            pl.BlockSpec((_BM, f), lambda i: (i, 0)),
            pl.BlockSpec((_BM, f), lambda i: (i, 0)),
            pl.BlockSpec((_BM, f), lambda i: (i, 0)),
            pl.BlockSpec((_BM, 1), lambda i: (i, 0)),
            pl.BlockSpec((1, f), lambda i: (0, 0)),
            pl.BlockSpec((f, f), lambda i: (0, 0)),
        ],
        out_specs=pl.BlockSpec((_BM, f), lambda i: (i, 0)),
        out_shape=jax.ShapeDtypeStruct((n, f), jnp.float32),
    )(p0, p1, hp1, dinv, b1, w2)


def _tc3(q0, q1, hp2, dinv, b2, bat3, wl, bl):
    n, f = hp2.shape
    g = 64
    out_dim = wl.shape[1]
    nb = n // _BM

    def body(q0_ref, q1_ref, hp_ref, dinv_ref, b2_ref, bat_ref, wl_ref,
             bl_ref, out_ref, pooled, cnt):
        i = pl.program_id(0)

        @pl.when(i == 0)
        def _():
            pooled[...] = jnp.zeros_like(pooled)
            cnt[...] = jnp.zeros_like(cnt)

        dq = dinv_ref[...] * (q0_ref[...] + q1_ref[...] + hp_ref[...])
        b = bat_ref[...].reshape(1, _BM)
        gids = lax.broadcasted_iota(jnp.int32, (g, 1), 0)
        mask = (b == gids).astype(jnp.float32)
        pooled[...] += jnp.dot(mask, dq, preferred_element_type=jnp.float32)
        cnt[...] += jnp.sum(mask, axis=1, keepdims=True)

        @pl.when(i == nb - 1)
        def _():
            pm = pooled[...] / jnp.maximum(cnt[...], 1.0) + b2_ref[...]
            out_ref[...] = jnp.dot(
                pm, wl_ref[...], preferred_element_type=jnp.float32
            ) + bl_ref[...]

    return pl.pallas_call(
        body,
        grid=(nb,),
        in_specs=[
            pl.BlockSpec((_BM, f), lambda i: (i, 0)),
            pl.BlockSpec((_BM, f), lambda i: (i, 0)),
            pl.BlockSpec((_BM, f), lambda i: (i, 0)),
            pl.BlockSpec((_BM, 1), lambda i: (i, 0)),
            pl.BlockSpec((1, f), lambda i: (0, 0)),
            pl.BlockSpec((1, 1, _BM), lambda i: (i, 0, 0)),
            pl.BlockSpec((f, out_dim), lambda i: (0, 0)),
            pl.BlockSpec((1, out_dim), lambda i: (0, 0)),
        ],
        out_specs=pl.BlockSpec((g, out_dim), lambda i: (0, 0)),
        out_shape=jax.ShapeDtypeStruct((g, out_dim), jnp.float32),
        scratch_shapes=[
            pltpu.VMEM((g, f), jnp.float32),
            pltpu.VMEM((g, 1), jnp.float32),
        ],
    )(q0, q1, hp2, dinv, b2, bat3, wl, bl)


def kernel(x, edge_index, batch, W1, b1, W2, b2, Wl, bl):
    n, f = x.shape
    e = edge_index.shape[1]
    nw = _NC * _NS
    ep = nw * _KO * _KI * _CH
    pad_i = jnp.arange(ep - e, dtype=edge_index.dtype) % _PAD
    src_p = jnp.concatenate([edge_index[0], pad_i])
    dst_p = jnp.concatenate([edge_index[1], pad_i + n])
    src2 = src_p.reshape(nw, _KO, _KI, _CH)
    dst2 = dst_p.reshape(nw, _KO, _KI, _CH)

    degp = _make_deg(n, e)(dst2).reshape(_NC, n)
    d0 = degp[0].reshape(n, 1)
    d1 = degp[1].reshape(n, 1)

    hp1, dinv = _tc1(x, W1, d0, d1)
    p = _make_agg(n, e, f)(hp1, src2, dst2)
    hp2 = _tc2(p[0], p[1], hp1, dinv, b1.reshape(1, f), W2)
    q = _make_agg(n, e, f)(hp2, src2, dst2)
    bat3 = batch.reshape(n // _BM, 1, _BM)
    return _tc3(q[0], q[1], hp2, dinv, b2.reshape(1, f),
                bat3, Wl, bl.reshape(1, -1))


# speedup vs baseline: 1.1214x; 1.1214x over previous
import functools

import jax
import jax.numpy as jnp
from jax import lax
from jax.experimental import pallas as pl
from jax.experimental.pallas import tpu as pltpu
from jax.experimental.pallas import tpu_sc as plsc

_NC = 2
_NS = 16
_CH = 64
_KI = 40
_KO = 4
_NBUF = 4
_PAD = 64


@functools.cache
def _make_deg(n, e):
    nw = _NC * _NS
    per_tile = e // nw
    nch = per_tile // _CH
    npad = ((n + nw * 16 - 1) // (nw * 16)) * (nw * 16)
    zseg = npad // _NS
    wb = 1000
    nwb = n // wb
    mesh = plsc.VectorSubcoreMesh(core_axis_name="c", subcore_axis_name="s")

    @functools.partial(
        pl.kernel,
        out_type=jax.ShapeDtypeStruct((_NC * n,), jnp.float32),
        mesh=mesh,
        scratch_types=[
            pltpu.VMEM((_KO, _KI, _CH), jnp.int32),
            pltpu.VMEM((_CH,), jnp.float32),
            pltpu.VMEM((zseg,), jnp.float32),
            pltpu.VMEM((wb,), jnp.float32),
            pltpu.VMEM_SHARED((npad,), jnp.float32),
        ],
    )
    def deg(dst_hbm, out_hbm, dst_v, ones_v, zbuf, wbuf, acc):
        c = lax.axis_index("c")
        s = lax.axis_index("s")
        w = s * _NC + c
        for j in range(_CH // 16):
            ones_v[pl.ds(j * 16, 16)] = jnp.ones((16,), jnp.float32)

        def zel(i, carry):
            zbuf[pl.ds(i * 16, 16)] = jnp.zeros((16,), jnp.float32)
            return carry

        lax.fori_loop(0, zseg // 16, zel, 0)
        pltpu.sync_copy(zbuf, acc.at[pl.ds(s * zseg, zseg)])
        plsc.subcore_barrier()
        pltpu.sync_copy(dst_hbm.at[w], dst_v)

        def body(k, carry):
            def inner(j, carry2):
                pltpu.sync_copy(ones_v, acc.at[dst_v.at[k, j]], add=True)
                return carry2

            return lax.fori_loop(0, _KI, inner, carry)

        lax.fori_loop(0, _KO, body, 0)
        plsc.subcore_barrier()

        @pl.when(s < nwb)
        def _():
            pltpu.sync_copy(acc.at[pl.ds(s * wb, wb)], wbuf)
            pltpu.sync_copy(wbuf, out_hbm.at[pl.ds(c * n + s * wb, wb)])

    return deg


@functools.cache
def _make_agg(n, e, f):
    zr = 40
    wb = 1000
    nwb = n // wb
    mesh = plsc.VectorSubcoreMesh(core_axis_name="c", subcore_axis_name="s")

    scratch = [
        pltpu.VMEM((1, _KI, _CH), jnp.int32),
        pltpu.VMEM((1, _KI, _CH), jnp.int32),
    ]
    scratch += [pltpu.VMEM((_CH, f), jnp.float32) for _ in range(_NBUF)]
    scratch += [
        pltpu.VMEM((zr, f), jnp.float32),
        pltpu.VMEM_SHARED((n + _PAD, f), jnp.float32),
    ]
    scratch += [pltpu.SemaphoreType.DMA for _ in range(_NBUF)]

    @functools.partial(
        pl.kernel,
        out_type=jax.ShapeDtypeStruct((_NC, n, f), jnp.float32),
        mesh=mesh,
        scratch_types=scratch,
    )
    def agg(hp_hbm, src_hbm, dst_hbm, out_hbm, src_v, dst_v, r0, r1, r2, r3,
            zbuf, acc, g0, g1, g2, g3):
        rows = (r0, r1, r2, r3)
        gsem = (g0, g1, g2, g3)
        c = lax.axis_index("c")
        s = lax.axis_index("s")
        w = s * _NC + c

        def zrow(r, carry):
            for j in range(f // 16):
                zbuf[r, pl.ds(j * 16, 16)] = jnp.zeros((16,), jnp.float32)
            return carry

        lax.fori_loop(0, zr, zrow, 0)

        nblk = n // zr

        def zcp(i, carry):
            blk_i = s + i * _NS

            @pl.when(blk_i < nblk)
            def _():
                pltpu.sync_copy(zbuf, acc.at[pl.ds(blk_i * zr, zr)])

            return carry

        lax.fori_loop(0, (nblk + _NS - 1) // _NS, zcp, 0)

        plsc.subcore_barrier()

        ngrp = _KI // _NBUF

        def blk(k, carry):
            pltpu.sync_copy(src_hbm.at[w, pl.ds(k, 1)], src_v)
            pltpu.sync_copy(dst_hbm.at[w, pl.ds(k, 1)], dst_v)
            for b in range(_NBUF):
                pltpu.async_copy(hp_hbm.at[src_v.at[0, jnp.int32(b)]],
                                 rows[b], gsem[b])

            def body(i, carry2):
                for b in range(_NBUF):
                    j = _NBUF * i + b
                    pltpu.make_async_copy(hp_hbm.at[src_v.at[0, j]], rows[b],
                                          gsem[b]).wait()
                    pltpu.sync_copy(rows[b], acc.at[dst_v.at[0, j]], add=True)
                    pltpu.async_copy(hp_hbm.at[src_v.at[0, j + _NBUF]],
                                     rows[b], gsem[b])
                return carry2

            lax.fori_loop(0, ngrp - 1, body, 0)
            for b in range(_NBUF):
                j = jnp.int32(_KI - _NBUF + b)
                pltpu.make_async_copy(hp_hbm.at[src_v.at[0, j]], rows[b],
                                      gsem[b]).wait()
                pltpu.sync_copy(rows[b], acc.at[dst_v.at[0, j]], add=True)
            return carry

        lax.fori_loop(0, _KO, blk, 0)

        plsc.subcore_barrier()

        def wcp(i, carry):
            blk_i = s + i * _NS

            @pl.when(blk_i < nblk)
            def _():
                pltpu.sync_copy(acc.at[pl.ds(blk_i * zr, zr)], zbuf)
                pltpu.sync_copy(zbuf, out_hbm.at[c, pl.ds(blk_i * zr, zr)])

            return carry

        lax.fori_loop(0, (nblk + _NS - 1) // _NS, wcp, 0)

    return agg


_BM = 1000


def _tc1(x, w1, d0, d1):
    n, f = x.shape

    def body(x_ref, w_ref, d0_ref, d1_ref, hp_ref, dinv_ref):
        dinv = lax.rsqrt(d0_ref[...] + d1_ref[...] + 1.0)
        h = jnp.dot(x_ref[...], w_ref[...], preferred_element_type=jnp.float32)
        hp_ref[...] = h * dinv
        dinv_ref[...] = dinv

    return pl.pallas_call(
        body,
        grid=(n // _BM,),
        in_specs=[
            pl.BlockSpec((_BM, f), lambda i: (i, 0)),
            pl.BlockSpec((f, f), lambda i: (0, 0)),
            pl.BlockSpec((_BM, 1), lambda i: (i, 0)),
            pl.BlockSpec((_BM, 1), lambda i: (i, 0)),
        ],
        out_specs=[
            pl.BlockSpec((_BM, f), lambda i: (i, 0)),
            pl.BlockSpec((_BM, 1), lambda i: (i, 0)),
        ],
        out_shape=[
            jax.ShapeDtypeStruct((n, f), jnp.float32),
            jax.ShapeDtypeStruct((n, 1), jnp.float32),
        ],
    )(x, w1, d0, d1)


def _tc2(p0, p1, hp1, dinv, b1, w2):
    n, f = hp1.shape

    def body(p0_ref, p1_ref, hp_ref, dinv_ref, b_ref, w_ref, out_ref):
        q = p0_ref[...] + p1_ref[...] + hp_ref[...]
        z = jnp.maximum(dinv_ref[...] * q + b_ref[...], 0.0)
        out_ref[...] = dinv_ref[...] * jnp.dot(
            z, w_ref[...], preferred_element_type=jnp.float32)

    return pl.pallas_call(
        body,
        grid=(n // _BM,),
        in_specs=[
            pl.BlockSpec((_BM, f), lambda i: (i, 0)),
            pl.BlockSpec((_BM, f), lambda i: (i, 0)),
            pl.BlockSpec((_BM, f), lambda i: (i, 0)),
            pl.BlockSpec((_BM, 1), lambda i: (i, 0)),
            pl.BlockSpec((1, f), lambda i: (0, 0)),
            pl.BlockSpec((f, f), lambda i: (0, 0)),
        ],
        out_specs=pl.BlockSpec((_BM, f), lambda i: (i, 0)),
        out_shape=jax.ShapeDtypeStruct((n, f), jnp.float32),
    )(p0, p1, hp1, dinv, b1, w2)


def _tc3(q0, q1, hp2, dinv, b2, bat3, wl, bl):
    n, f = hp2.shape
    g = 64
    out_dim = wl.shape[1]
    nb = n // _BM

    def body(q0_ref, q1_ref, hp_ref, dinv_ref, b2_ref, bat_ref, wl_ref,
             bl_ref, out_ref, pooled, cnt):
        i = pl.program_id(0)

        @pl.when(i == 0)
        def _():
            pooled[...] = jnp.zeros_like(pooled)
            cnt[...] = jnp.zeros_like(cnt)

        dq = dinv_ref[...] * (q0_ref[...] + q1_ref[...] + hp_ref[...])
        b = bat_ref[...].reshape(1, _BM)
        gids = lax.broadcasted_iota(jnp.int32, (g, 1), 0)
        mask = (b == gids).astype(jnp.float32)
        pooled[...] += jnp.dot(mask, dq, preferred_element_type=jnp.float32)
        cnt[...] += jnp.sum(mask, axis=1, keepdims=True)

        @pl.when(i == nb - 1)
        def _():
            pm = pooled[...] / jnp.maximum(cnt[...], 1.0) + b2_ref[...]
            out_ref[...] = jnp.dot(
                pm, wl_ref[...], preferred_element_type=jnp.float32
            ) + bl_ref[...]

    return pl.pallas_call(
        body,
        grid=(nb,),
        in_specs=[
            pl.BlockSpec((_BM, f), lambda i: (i, 0)),
            pl.BlockSpec((_BM, f), lambda i: (i, 0)),
            pl.BlockSpec((_BM, f), lambda i: (i, 0)),
            pl.BlockSpec((_BM, 1), lambda i: (i, 0)),
            pl.BlockSpec((1, f), lambda i: (0, 0)),
            pl.BlockSpec((1, 1, _BM), lambda i: (i, 0, 0)),
            pl.BlockSpec((f, out_dim), lambda i: (0, 0)),
            pl.BlockSpec((1, out_dim), lambda i: (0, 0)),
        ],
        out_specs=pl.BlockSpec((g, out_dim), lambda i: (0, 0)),
        out_shape=jax.ShapeDtypeStruct((g, out_dim), jnp.float32),
        scratch_shapes=[
            pltpu.VMEM((g, f), jnp.float32),
            pltpu.VMEM((g, 1), jnp.float32),
        ],
    )(q0, q1, hp2, dinv, b2, bat3, wl, bl)


def kernel(x, edge_index, batch, W1, b1, W2, b2, Wl, bl):
    n, f = x.shape
    e = edge_index.shape[1]
    nw = _NC * _NS
    ep = nw * _KO * _KI * _CH
    pad_i = jnp.arange(ep - e, dtype=edge_index.dtype) % _PAD
    src_p = jnp.concatenate([edge_index[0], pad_i])
    dst_p = jnp.concatenate([edge_index[1], pad_i + n])
    src2 = src_p.reshape(nw, _KO, _KI, _CH)
    dst2 = dst_p.reshape(nw, _KO, _KI, _CH)

    degp = _make_deg(n, e)(dst2).reshape(_NC, n)
    d0 = degp[0].reshape(n, 1)
    d1 = degp[1].reshape(n, 1)

    hp1, dinv = _tc1(x, W1, d0, d1)
    p = _make_agg(n, e, f)(hp1, src2, dst2)
    hp2 = _tc2(p[0], p[1], hp1, dinv, b1.reshape(1, f), W2)
    q = _make_agg(n, e, f)(hp2, src2, dst2)
    bat3 = batch.reshape(n // _BM, 1, _BM)
    return _tc3(q[0], q[1], hp2, dinv, b2.reshape(1, f),
                bat3, Wl, bl.reshape(1, -1))


# speedup vs baseline: 1.1279x; 1.0058x over previous
import functools

import jax
import jax.numpy as jnp
from jax import lax
from jax.experimental import pallas as pl
from jax.experimental.pallas import tpu as pltpu
from jax.experimental.pallas import tpu_sc as plsc

_NC = 2
_NS = 16
_CH = 64
_KI = 40
_KO = 4
_NBUF = 4
_PAD = 64


@functools.cache
def _make_deg(n, e):
    nw = _NC * _NS
    per_tile = e // nw
    nch = per_tile // _CH
    npad = ((n + nw * 16 - 1) // (nw * 16)) * (nw * 16)
    zseg = npad // _NS
    wb = 1000
    nwb = n // wb
    mesh = plsc.VectorSubcoreMesh(core_axis_name="c", subcore_axis_name="s")

    @functools.partial(
        pl.kernel,
        out_type=jax.ShapeDtypeStruct((_NC * n,), jnp.float32),
        mesh=mesh,
        scratch_types=[
            pltpu.VMEM((_KO, _KI, _CH), jnp.int32),
            pltpu.VMEM((_CH,), jnp.float32),
            pltpu.VMEM((zseg,), jnp.float32),
            pltpu.VMEM((wb,), jnp.float32),
            pltpu.VMEM_SHARED((npad,), jnp.float32),
        ],
    )
    def deg(dst_hbm, out_hbm, dst_v, ones_v, zbuf, wbuf, acc):
        c = lax.axis_index("c")
        s = lax.axis_index("s")
        w = s * _NC + c
        for j in range(_CH // 16):
            ones_v[pl.ds(j * 16, 16)] = jnp.ones((16,), jnp.float32)

        def zel(i, carry):
            zbuf[pl.ds(i * 16, 16)] = jnp.zeros((16,), jnp.float32)
            return carry

        lax.fori_loop(0, zseg // 16, zel, 0)
        pltpu.sync_copy(zbuf, acc.at[pl.ds(s * zseg, zseg)])
        plsc.subcore_barrier()
        pltpu.sync_copy(dst_hbm.at[w], dst_v)

        def body(k, carry):
            def inner(j, carry2):
                pltpu.sync_copy(ones_v, acc.at[dst_v.at[k, j]], add=True)
                return carry2

            return lax.fori_loop(0, _KI, inner, carry)

        lax.fori_loop(0, _KO, body, 0)
        plsc.subcore_barrier()

        @pl.when(s < nwb)
        def _():
            pltpu.sync_copy(acc.at[pl.ds(s * wb, wb)], wbuf)
            pltpu.sync_copy(wbuf, out_hbm.at[pl.ds(c * n + s * wb, wb)])

    return deg


@functools.cache
def _make_agg(n, e, f):
    zr = 16
    wb = 1000
    nwb = n // wb
    mesh = plsc.VectorSubcoreMesh(core_axis_name="c", subcore_axis_name="s")

    scratch = [
        pltpu.VMEM((2, _KI, _CH), jnp.int32),
        pltpu.VMEM((1, _KI, _CH), jnp.int32),
    ]
    scratch += [pltpu.VMEM((_CH, f), jnp.float32) for _ in range(_NBUF)]
    scratch += [
        pltpu.VMEM((zr, f), jnp.float32),
        pltpu.VMEM_SHARED((n + _PAD, f), jnp.float32),
    ]
    scratch += [pltpu.SemaphoreType.DMA for _ in range(_NBUF)]

    @functools.partial(
        pl.kernel,
        out_type=jax.ShapeDtypeStruct((_NC, n, f), jnp.float32),
        mesh=mesh,
        scratch_types=scratch,
    )
    def agg(hp_hbm, src_hbm, dst_hbm, out_hbm, src_v, dst_v, r0, r1, r2, r3,
            zbuf, acc, g0, g1, g2, g3):
        rows = (r0, r1, r2, r3)
        gsem = (g0, g1, g2, g3)
        c = lax.axis_index("c")
        s = lax.axis_index("s")
        w = s * _NC + c

        def zrow(r, carry):
            for j in range(f // 16):
                zbuf[r, pl.ds(j * 16, 16)] = jnp.zeros((16,), jnp.float32)
            return carry

        lax.fori_loop(0, zr, zrow, 0)

        nblk = n // zr

        def zcp(i, carry):
            blk_i = s + i * _NS

            @pl.when(blk_i < nblk)
            def _():
                pltpu.sync_copy(zbuf, acc.at[pl.ds(blk_i * zr, zr)])

            return carry

        lax.fori_loop(0, (nblk + _NS - 1) // _NS, zcp, 0)

        ngrp = _KI // _NBUF
        pltpu.sync_copy(src_hbm.at[w, pl.ds(0, 1)], src_v.at[pl.ds(0, 1)])
        for b in range(_NBUF):
            pltpu.async_copy(hp_hbm.at[src_v.at[0, jnp.int32(b)]],
                             rows[b], gsem[b])

        plsc.subcore_barrier()

        def blk(k, carry):
            p = lax.rem(k, 2)
            pltpu.sync_copy(dst_hbm.at[w, pl.ds(k, 1)], dst_v)

            def body(i, carry2):
                for b in range(_NBUF):
                    j = _NBUF * i + b
                    pltpu.make_async_copy(hp_hbm.at[src_v.at[p, j]], rows[b],
                                          gsem[b]).wait()
                    pltpu.sync_copy(rows[b], acc.at[dst_v.at[0, j]], add=True)
                    pltpu.async_copy(hp_hbm.at[src_v.at[p, j + _NBUF]],
                                     rows[b], gsem[b])
                return carry2

            lax.fori_loop(0, ngrp - 1, body, 0)

            @pl.when(k < _KO - 1)
            def _():
                pltpu.sync_copy(src_hbm.at[w, pl.ds(k + 1, 1)],
                                src_v.at[pl.ds(1 - p, 1)])

            for b in range(_NBUF):
                j = jnp.int32(_KI - _NBUF + b)
                pltpu.make_async_copy(hp_hbm.at[src_v.at[p, j]], rows[b],
                                      gsem[b]).wait()
                pltpu.sync_copy(rows[b], acc.at[dst_v.at[0, j]], add=True)

                @pl.when(k < _KO - 1)
                def _():
                    pltpu.async_copy(
                        hp_hbm.at[src_v.at[1 - p, jnp.int32(b)]],
                        rows[b], gsem[b])

            return carry

        lax.fori_loop(0, _KO, blk, 0)

        plsc.subcore_barrier()

        def wcp(i, carry):
            blk_i = s + i * _NS

            @pl.when(blk_i < nblk)
            def _():
                pltpu.sync_copy(acc.at[pl.ds(blk_i * zr, zr)], zbuf)
                pltpu.sync_copy(zbuf, out_hbm.at[c, pl.ds(blk_i * zr, zr)])

            return carry

        lax.fori_loop(0, (nblk + _NS - 1) // _NS, wcp, 0)

    return agg


_BM = 1000


def _tc1(x, w1, d0, d1):
    n, f = x.shape

    def body(x_ref, w_ref, d0_ref, d1_ref, hp_ref, dinv_ref):
        dinv = lax.rsqrt(d0_ref[...] + d1_ref[...] + 1.0)
        h = jnp.dot(x_ref[...], w_ref[...], preferred_element_type=jnp.float32)
        hp_ref[...] = h * dinv
        dinv_ref[...] = dinv

    return pl.pallas_call(
        body,
        grid=(n // _BM,),
        in_specs=[
            pl.BlockSpec((_BM, f), lambda i: (i, 0)),
            pl.BlockSpec((f, f), lambda i: (0, 0)),
            pl.BlockSpec((_BM, 1), lambda i: (i, 0)),
            pl.BlockSpec((_BM, 1), lambda i: (i, 0)),
        ],
        out_specs=[
            pl.BlockSpec((_BM, f), lambda i: (i, 0)),
            pl.BlockSpec((_BM, 1), lambda i: (i, 0)),
        ],
        out_shape=[
            jax.ShapeDtypeStruct((n, f), jnp.float32),
            jax.ShapeDtypeStruct((n, 1), jnp.float32),
        ],
    )(x, w1, d0, d1)


def _tc2(p0, p1, hp1, dinv, b1, w2):
    n, f = hp1.shape

    def body(p0_ref, p1_ref, hp_ref, dinv_ref, b_ref, w_ref, out_ref):
        q = p0_ref[...] + p1_ref[...] + hp_ref[...]
        z = jnp.maximum(dinv_ref[...] * q + b_ref[...], 0.0)
        out_ref[...] = dinv_ref[...] * jnp.dot(
            z, w_ref[...], preferred_element_type=jnp.float32)

    return pl.pallas_call(
        body,
        grid=(n // _BM,),
        in_specs=[
            pl.BlockSpec((_BM, f), lambda i: (i, 0)),
            pl.BlockSpec((_BM, f), lambda i: (i, 0)),
            pl.BlockSpec((_BM, f), lambda i: (i, 0)),
            pl.BlockSpec((_BM, 1), lambda i: (i, 0)),
            pl.BlockSpec((1, f), lambda i: (0, 0)),
            pl.BlockSpec((f, f), lambda i: (0, 0)),
        ],
        out_specs=pl.BlockSpec((_BM, f), lambda i: (i, 0)),
        out_shape=jax.ShapeDtypeStruct((n, f), jnp.float32),
    )(p0, p1, hp1, dinv, b1, w2)


def _tc3(q0, q1, hp2, dinv, b2, bat3, wl, bl):
    n, f = hp2.shape
    g = 64
    out_dim = wl.shape[1]
    nb = n // _BM

    def body(q0_ref, q1_ref, hp_ref, dinv_ref, b2_ref, bat_ref, wl_ref,
             bl_ref, out_ref, pooled, cnt):
        i = pl.program_id(0)

        @pl.when(i == 0)
        def _():
            pooled[...] = jnp.zeros_like(pooled)
            cnt[...] = jnp.zeros_like(cnt)

        dq = dinv_ref[...] * (q0_ref[...] + q1_ref[...] + hp_ref[...])
        b = bat_ref[...].reshape(1, _BM)
        gids = lax.broadcasted_iota(jnp.int32, (g, 1), 0)
        mask = (b == gids).astype(jnp.float32)
        pooled[...] += jnp.dot(mask, dq, preferred_element_type=jnp.float32)
        cnt[...] += jnp.sum(mask, axis=1, keepdims=True)

        @pl.when(i == nb - 1)
        def _():
            pm = pooled[...] / jnp.maximum(cnt[...], 1.0) + b2_ref[...]
            out_ref[...] = jnp.dot(
                pm, wl_ref[...], preferred_element_type=jnp.float32
            ) + bl_ref[...]

    return pl.pallas_call(
        body,
        grid=(nb,),
        in_specs=[
            pl.BlockSpec((_BM, f), lambda i: (i, 0)),
            pl.BlockSpec((_BM, f), lambda i: (i, 0)),
            pl.BlockSpec((_BM, f), lambda i: (i, 0)),
            pl.BlockSpec((_BM, 1), lambda i: (i, 0)),
            pl.BlockSpec((1, f), lambda i: (0, 0)),
            pl.BlockSpec((1, 1, _BM), lambda i: (i, 0, 0)),
            pl.BlockSpec((f, out_dim), lambda i: (0, 0)),
            pl.BlockSpec((1, out_dim), lambda i: (0, 0)),
        ],
        out_specs=pl.BlockSpec((g, out_dim), lambda i: (0, 0)),
        out_shape=jax.ShapeDtypeStruct((g, out_dim), jnp.float32),
        scratch_shapes=[
            pltpu.VMEM((g, f), jnp.float32),
            pltpu.VMEM((g, 1), jnp.float32),
        ],
    )(q0, q1, hp2, dinv, b2, bat3, wl, bl)


def kernel(x, edge_index, batch, W1, b1, W2, b2, Wl, bl):
    n, f = x.shape
    e = edge_index.shape[1]
    nw = _NC * _NS
    ep = nw * _KO * _KI * _CH
    pad_i = jnp.arange(ep - e, dtype=edge_index.dtype) % _PAD
    src_p = jnp.concatenate([edge_index[0], pad_i])
    dst_p = jnp.concatenate([edge_index[1], pad_i + n])
    src2 = src_p.reshape(nw, _KO, _KI, _CH)
    dst2 = dst_p.reshape(nw, _KO, _KI, _CH)

    degp = _make_deg(n, e)(dst2).reshape(_NC, n)
    d0 = degp[0].reshape(n, 1)
    d1 = degp[1].reshape(n, 1)

    hp1, dinv = _tc1(x, W1, d0, d1)
    p = _make_agg(n, e, f)(hp1, src2, dst2)
    hp2 = _tc2(p[0], p[1], hp1, dinv, b1.reshape(1, f), W2)
    q = _make_agg(n, e, f)(hp2, src2, dst2)
    bat3 = batch.reshape(n // _BM, 1, _BM)
    return _tc3(q[0], q[1], hp2, dinv, b2.reshape(1, f),
                bat3, Wl, bl.reshape(1, -1))
